# Initial kernel scaffold; baseline (speedup 1.0000x reference)
import jax
import jax.numpy as jnp
from jax.experimental import pallas as pl


def kernel(x, t, c_vector, W_msg, b_msg, W_time, W_ctx, W_out, b_out):
    raise NotImplementedError("write your pallas kernel here")


# speedup vs baseline: 43.3674x; 43.3674x over previous
import math

import jax
import jax.numpy as jnp
from jax.experimental import pallas as pl

K = 16
R = 256


def _fused_kernel(x_ref, t_ref, c_ref, wmsg_ref, bmsg_ref, wtime_ref,
                  wctx_ref, wout_ref, bout_ref, out_ref):
    r = pl.program_id(1)
    xs = x_ref[0]
    n_nodes = xs.shape[0]

    half = wtime_ref.shape[0] // 2
    freq_i = jax.lax.broadcasted_iota(jnp.int32, (1, half), 1).astype(
        jnp.float32)
    freqs = jnp.exp(freq_i * (-math.log(10000.0) / half))
    args = t_ref[0] * freqs
    temb = jnp.concatenate([jnp.cos(args), jnp.sin(args)], axis=-1)
    cond = (jnp.dot(temb, wtime_ref[...], preferred_element_type=jnp.float32)
            + jnp.dot(c_ref[0], wctx_ref[...],
                      preferred_element_type=jnp.float32))

    xr = x_ref[0, pl.ds(r * R, R), :]
    sq_full = jnp.sum(xs * xs, axis=1)
    sq_r = jnp.sum(xr * xr, axis=1)
    cross = jax.lax.dot_general(
        xr, xs, (((1,), (1,)), ((), ())),
        preferred_element_type=jnp.float32)
    d2 = sq_r[:, None] - 2.0 * cross + sq_full[None, :]
    rows = jax.lax.broadcasted_iota(jnp.int32, (R, n_nodes), 0) + r * R
    cols = jax.lax.broadcasted_iota(jnp.int32, (R, n_nodes), 1)
    d2 = jnp.where(rows == cols, d2 + 1e9, d2)

    w = d2
    m = None
    for _ in range(K):
        m = jnp.min(w, axis=1, keepdims=True)
        w = jnp.where(w == m, jnp.inf, w)
    mask = (d2 <= m).astype(jnp.float32)

    nbr = jnp.dot(mask, xs, preferred_element_type=jnp.float32)
    agg = (jnp.dot(nbr, wmsg_ref[...], preferred_element_type=jnp.float32)
           + float(K) * bmsg_ref[...][None, :])
    h = jnp.maximum(xr + agg + cond, 0.0)
    out_ref[0] = (jnp.dot(h, wout_ref[...], preferred_element_type=jnp.float32)
                  + bout_ref[...][None, :])


def kernel(x, t, c_vector, W_msg, b_msg, W_time, W_ctx, W_out, b_out):
    B, N, D = x.shape
    CTX = c_vector.shape[1]
    nb = N // R
    grid = (B, nb)
    out = pl.pallas_call(
        _fused_kernel,
        grid=grid,
        in_specs=[
            pl.BlockSpec((1, N, D), lambda b, r: (b, 0, 0)),
            pl.BlockSpec((1, 1, 1), lambda b, r: (b, 0, 0)),
            pl.BlockSpec((1, 1, CTX), lambda b, r: (b, 0, 0)),
            pl.BlockSpec((D, D), lambda b, r: (0, 0)),
            pl.BlockSpec((D,), lambda b, r: (0,)),
            pl.BlockSpec((D, D), lambda b, r: (0, 0)),
            pl.BlockSpec((CTX, D), lambda b, r: (0, 0)),
            pl.BlockSpec((D, D), lambda b, r: (0, 0)),
            pl.BlockSpec((D,), lambda b, r: (0,)),
        ],
        out_specs=pl.BlockSpec((1, R, D), lambda b, r: (b, r, 0)),
        out_shape=jax.ShapeDtypeStruct((B, N, D), jnp.float32),
    )(x, t.reshape(B, 1, 1), c_vector.reshape(B, 1, CTX), W_msg, b_msg,
      W_time, W_ctx, W_out, b_out)
    return out
